# baseline (speedup 1.0000x reference)
import functools

import jax
import jax.numpy as jnp
from jax import lax
from jax.experimental import pallas as pl
from jax.experimental.pallas import tpu as pltpu
from jax.experimental.pallas import tpu_sc as plsc

N = 10000
E = 320000
D = 128
NH = 8
DH = 16
RT = 4
G = 100

NC = 2
NS = 16
NW = NC * NS
C = 80
NCHUNK = E // C
NJ = NCHUNK // NW
GP = 112


def _shuf(v, idx):
    return v.at[idx].get(mode="promise_in_bounds")


_mesh = plsc.VectorSubcoreMesh(core_axis_name="c", subcore_axis_name="s")


def _prep_body(h_ref, wqp_ref, wkv_ref, qs_ref, p_ref, k_ref, v_ref):
    hb = h_ref[...]
    dtab = jnp.dot(hb, wqp_ref[...], preferred_element_type=jnp.float32)
    stab = jnp.dot(hb, wkv_ref[...], preferred_element_type=jnp.float32)
    qs_ref[...] = dtab[:, :D]
    p_ref[...] = dtab[:, D:]
    k_ref[...] = stab[:, :D]
    v_ref[...] = stab[:, D:]


_prep_call = pl.pallas_call(
    _prep_body,
    grid=(10,),
    in_specs=[
        pl.BlockSpec((N // 10, D), lambda i: (i, 0)),
        pl.BlockSpec((D, 2 * D), lambda i: (0, 0)),
        pl.BlockSpec((D, 2 * D), lambda i: (0, 0)),
    ],
    out_specs=[pl.BlockSpec((N // 10, D), lambda i: (i, 0))] * 4,
    out_shape=[jax.ShapeDtypeStruct((N, D), jnp.float32)] * 4,
)


@functools.partial(
    pl.kernel,
    out_type=[
        jax.ShapeDtypeStruct((NC, N, 16), jnp.float32),
        jax.ShapeDtypeStruct((E * 16,), jnp.float32),
    ],
    mesh=_mesh,
    compiler_params=pltpu.CompilerParams(use_tc_tiling_on_sc=False),
    scratch_types=[
        pltpu.VMEM_SHARED((N, 16), jnp.float32),
        pltpu.VMEM((C,), jnp.int32), pltpu.VMEM((C,), jnp.int32),
        pltpu.VMEM((C,), jnp.int32), pltpu.VMEM((C,), jnp.int32),
        pltpu.VMEM((C,), jnp.int32), pltpu.VMEM((C,), jnp.int32),
        pltpu.VMEM((C * DH,), jnp.float32), pltpu.VMEM((C * DH,), jnp.float32),
        pltpu.VMEM((C, D), jnp.float32), pltpu.VMEM((C, D), jnp.float32),
        pltpu.VMEM((C, D), jnp.float32), pltpu.VMEM((C, D), jnp.float32),
        pltpu.VMEM((C, D), jnp.float32), pltpu.VMEM((C, D), jnp.float32),
        pltpu.VMEM((C * 16,), jnp.float32), pltpu.VMEM((C * 16,), jnp.float32),
        pltpu.VMEM((C, 16), jnp.float32),
        pltpu.VMEM((RT * 16,), jnp.float32),
        pltpu.SemaphoreType.DMA, pltpu.SemaphoreType.DMA,
        pltpu.SemaphoreType.DMA, pltpu.SemaphoreType.DMA,
        pltpu.SemaphoreType.DMA, pltpu.SemaphoreType.DMA,
    ],
)
def _score_kernel(qs_t, p_t, k_t, dst_h, src_h, et_h, ev_h, rel_h,
                  dn_out, ex_out,
                  dn_sh, dst0, dst1, src0, src1, et0, et1, ev0, ev1,
                  qs0, qs1, p0, p1, k0, k1, exf0, exf1, exs_b, rel_v,
                  semi0, semi1, semg0, semg1, semx0, semx1):
    cid = lax.axis_index("c")
    sid = lax.axis_index("s")
    wid = cid * NS + sid
    lanes = lax.iota(jnp.int32, 16)
    rot8 = (lanes + 8) & 15
    rot4 = (lanes + 4) & 15
    rot2 = (lanes + 2) & 15
    rot1 = (lanes + 1) & 15

    f4i = (lanes & 8) | ((lanes + 4) & 7)
    f2i = (lanes & 12) | ((lanes + 2) & 3)
    f1i = lanes ^ 1
    mAB = ((lanes & 4) << 1) | (lanes & 3)
    mCD = ((lanes & 6) << 1) | (lanes & 1)
    lo8 = lanes < 8

    def _hsum8(ps):
        f = [p + _shuf(p, rot8) for p in ps]
        m = [jnp.where(lo8, f[2 * a], f[2 * a + 1]) for a in range(4)]
        m = [v + _shuf(v, f4i) for v in m]
        q = [jnp.where(lo8, _shuf(m[2 * b], mAB), _shuf(m[2 * b + 1], mAB))
             for b in range(2)]
        q = [v + _shuf(v, f2i) for v in q]
        r = jnp.where(lo8, _shuf(q[0], mCD), _shuf(q[1], mCD))
        return r + _shuf(r, f1i)

    zv = jnp.zeros((16,), jnp.float32)

    def _zero_exs(i, _):
        exs_b[i, :] = zv
        return 0

    lax.fori_loop(0, C, _zero_exs, 0)
    zb = sid * 624
    for t in range(7):
        pltpu.sync_copy(exs_b.at[pl.ds(0, 80)], dn_sh.at[pl.ds(zb + t * 80, 80)])
    pltpu.sync_copy(exs_b.at[pl.ds(0, 64)], dn_sh.at[pl.ds(zb + 560, 64)])

    @pl.when(sid == NS - 1)
    def _zero_tail():
        pltpu.sync_copy(exs_b.at[pl.ds(0, 16)], dn_sh.at[pl.ds(NS * 624, 16)])

    pltpu.sync_copy(rel_h, rel_v)
    plsc.subcore_barrier()
    r0v = rel_v[pl.ds(0, 16)]
    r1v = rel_v[pl.ds(16, 16)]
    r2v = rel_v[pl.ds(32, 16)]
    r3v = rel_v[pl.ds(48, 16)]

    sets = [
        (dst0, src0, et0, ev0, qs0, p0, k0, exf0, semi0, semg0, semx0),
        (dst1, src1, et1, ev1, qs1, p1, k1, exf1, semi1, semg1, semx1),
    ]

    def _off(j):
        return (wid + jnp.minimum(j, NJ - 1) * NW) * C

    def _issue_idx(j, s):
        o = _off(j)
        pltpu.async_copy(dst_h.at[pl.ds(o, C)], s[0], s[8])
        pltpu.async_copy(src_h.at[pl.ds(o, C)], s[1], s[8])
        pltpu.async_copy(et_h.at[pl.ds(o, C)], s[2], s[8])
        pltpu.async_copy(ev_h.at[pl.ds(o * DH, C * DH)], s[3], s[8])

    def _drain_idx(j, s):
        o = _off(j)
        pltpu.make_async_copy(dst_h.at[pl.ds(o, C)], s[0], s[8]).wait()
        pltpu.make_async_copy(src_h.at[pl.ds(o, C)], s[1], s[8]).wait()
        pltpu.make_async_copy(et_h.at[pl.ds(o, C)], s[2], s[8]).wait()
        pltpu.make_async_copy(ev_h.at[pl.ds(o * DH, C * DH)], s[3], s[8]).wait()

    def _issue_gather(s):
        pltpu.async_copy(qs_t.at[s[0]], s[4], s[9])
        pltpu.async_copy(p_t.at[s[0]], s[5], s[9])
        pltpu.async_copy(k_t.at[s[1]], s[6], s[9])

    def _drain_gather(s):
        pltpu.make_async_copy(qs_t.at[s[0]], s[4], s[9]).wait()
        pltpu.make_async_copy(p_t.at[s[0]], s[5], s[9]).wait()
        pltpu.make_async_copy(k_t.at[s[1]], s[6], s[9]).wait()

    def _compute(s):
        dst_i, _, et_i, ev_b, qs_b, p_b, k_b, exf = s[:8]

        def _edges(i2, _):
            etv = et_i[pl.ds(i2 * 16, 16)]
            for u in range(16):
                e = i2 * 16 + u
                et = etv[u]
                exrel = jnp.where(et == 0, r0v,
                                  jnp.where(et == 1, r1v,
                                            jnp.where(et == 2, r2v, r3v)))
                evv = ev_b[pl.ds(e * DH, 16)]
                ps = []
                for g in range(NH):
                    q = qs_b[e, pl.ds(g * 16, 16)]
                    k = k_b[e, pl.ds(g * 16, 16)]
                    p2 = p_b[e, pl.ds(g * 16, 16)]
                    ps.append(q * k + evv * p2)
                ex = jnp.exp(_hsum8(ps)) * exrel
                exf[pl.ds(e * 16, 16)] = ex
                exs_b[e, :] = ex
            return 0

        lax.fori_loop(0, C // 16, _edges, 0)
        pltpu.sync_copy(exs_b, dn_sh.at[dst_i], add=True)

    def _half(j, jp, P, Q):
        _drain_idx(j + 1, Q)
        _issue_gather(Q)
        _drain_gather(P)

        @pl.when(jp)
        def _():
            o = _off(j - 2)
            pltpu.make_async_copy(P[7], ex_out.at[pl.ds(o * 16, C * 16)],
                                  P[10]).wait()

        _compute(P)
        pltpu.async_copy(P[7], ex_out.at[pl.ds(_off(j) * 16, C * 16)], P[10])
        _issue_idx(j + 2, P)

    _issue_idx(0, sets[0])
    _drain_idx(0, sets[0])
    _issue_gather(sets[0])
    _issue_idx(1, sets[1])

    def _body(j2, _):
        a = 2 * j2
        _half(a, j2 >= 1, sets[0], sets[1])
        _half(a + 1, j2 >= 1, sets[1], sets[0])
        return 0

    lax.fori_loop(0, (NJ - 1) // 2, _body, 0)

    _drain_idx(NJ, sets[1])
    _drain_gather(sets[0])
    o = _off(NJ - 3)
    pltpu.make_async_copy(sets[0][7], ex_out.at[pl.ds(o * 16, C * 16)],
                          sets[0][10]).wait()
    o = _off(NJ - 2)
    pltpu.make_async_copy(sets[1][7], ex_out.at[pl.ds(o * 16, C * 16)],
                          sets[1][10]).wait()
    _compute(sets[0])
    pltpu.sync_copy(sets[0][7], ex_out.at[pl.ds(_off(NJ - 1) * 16, C * 16)])

    plsc.subcore_barrier()

    @pl.when(sid == 0)
    def _copy_out():
        def _dncp(t, _):
            pltpu.sync_copy(dn_sh.at[pl.ds(t * 2000, 2000)],
                            dn_out.at[cid, pl.ds(t * 2000, 2000)])
            return 0

        lax.fori_loop(0, 5, _dncp, 0)


@functools.partial(
    pl.kernel,
    out_type=jax.ShapeDtypeStruct((NC, N, D), jnp.float32),
    mesh=_mesh,
    compiler_params=pltpu.CompilerParams(use_tc_tiling_on_sc=False),
    scratch_types=[
        pltpu.VMEM_SHARED((N, D), jnp.float32),
        pltpu.VMEM((C,), jnp.int32), pltpu.VMEM((C,), jnp.int32),
        pltpu.VMEM((C,), jnp.int32), pltpu.VMEM((C,), jnp.int32),
        pltpu.VMEM((C * 16,), jnp.float32), pltpu.VMEM((C * 16,), jnp.float32),
        pltpu.VMEM((C, D), jnp.float32), pltpu.VMEM((C, D), jnp.float32),
        pltpu.VMEM((C, D), jnp.float32),
        pltpu.SemaphoreType.DMA, pltpu.SemaphoreType.DMA,
        pltpu.SemaphoreType.DMA, pltpu.SemaphoreType.DMA,
    ],
)
def _agg_kernel(v_t, dst_h, src_h, exf_h,
                u_out,
                u_sh, dst0, dst1, src0, src1, exr0, exr1, v0, v1, msg_b,
                semi0, semi1, semg0, semg1):
    cid = lax.axis_index("c")
    sid = lax.axis_index("s")
    wid = cid * NS + sid

    zv = jnp.zeros((16,), jnp.float32)

    def _zero_msg(i, _):
        msg_b[i // 8, pl.ds((i % 8) * 16, 16)] = zv
        return 0

    lax.fori_loop(0, C * 8, _zero_msg, 0)
    zb = sid * 624
    for t in range(7):
        pltpu.sync_copy(msg_b.at[pl.ds(0, 80)], u_sh.at[pl.ds(zb + t * 80, 80)])
    pltpu.sync_copy(msg_b.at[pl.ds(0, 64)], u_sh.at[pl.ds(zb + 560, 64)])

    @pl.when(sid == NS - 1)
    def _zero_tail():
        pltpu.sync_copy(msg_b.at[pl.ds(0, 16)], u_sh.at[pl.ds(NS * 624, 16)])

    plsc.subcore_barrier()

    sets = [
        (dst0, src0, exr0, v0, semi0, semg0),
        (dst1, src1, exr1, v1, semi1, semg1),
    ]

    def _off(j):
        return (wid + jnp.minimum(j, NJ - 1) * NW) * C

    def _issue_idx(j, s):
        o = _off(j)
        pltpu.async_copy(dst_h.at[pl.ds(o, C)], s[0], s[4])
        pltpu.async_copy(src_h.at[pl.ds(o, C)], s[1], s[4])
        pltpu.async_copy(exf_h.at[pl.ds(o * 16, C * 16)], s[2], s[4])

    def _drain_idx(j, s):
        o = _off(j)
        pltpu.make_async_copy(dst_h.at[pl.ds(o, C)], s[0], s[4]).wait()
        pltpu.make_async_copy(src_h.at[pl.ds(o, C)], s[1], s[4]).wait()
        pltpu.make_async_copy(exf_h.at[pl.ds(o * 16, C * 16)], s[2], s[4]).wait()

    def _issue_gather(s):
        pltpu.async_copy(v_t.at[s[1]], s[3], s[5])

    def _drain_gather(s):
        pltpu.make_async_copy(v_t.at[s[1]], s[3], s[5]).wait()

    def _compute(s):
        dst_i, _, exr, v_b = s[:4]

        def _edges(i2, _):
            for u in range(16):
                e = i2 * 16 + u
                ex = exr[pl.ds(e * 16, 16)]
                for g in range(NH):
                    v = v_b[e, pl.ds(g * 16, 16)]
                    msg_b[e, pl.ds(g * 16, 16)] = ex[2 * g] * v
            return 0

        lax.fori_loop(0, C // 16, _edges, 0)
        pltpu.sync_copy(msg_b, u_sh.at[dst_i], add=True)

    def _half(j, P, Q):
        _drain_idx(j + 1, Q)
        _issue_gather(Q)
        _drain_gather(P)
        _compute(P)
        _issue_idx(j + 2, P)

    _issue_idx(0, sets[0])
    _drain_idx(0, sets[0])
    _issue_gather(sets[0])
    _issue_idx(1, sets[1])

    def _body(j2, _):
        a = 2 * j2
        _half(a, sets[0], sets[1])
        _half(a + 1, sets[1], sets[0])
        return 0

    lax.fori_loop(0, (NJ - 1) // 2, _body, 0)

    _drain_idx(NJ, sets[1])
    _drain_gather(sets[0])
    _compute(sets[0])

    plsc.subcore_barrier()

    @pl.when(sid == 0)
    def _copy_out():
        pltpu.sync_copy(u_sh, u_out.at[cid])


def _fin_body(u_ref, dn_ref, h_ref, wo_ref, em_ref, out_ref):
    u = u_ref[0] + u_ref[1]
    dn = dn_ref[0] + dn_ref[1]
    rec = 1.0 / (dn + 1e-16)
    scale = jnp.dot(rec, em_ref[...], preferred_element_type=jnp.float32)
    z = jnp.dot(u * scale, wo_ref[...], preferred_element_type=jnp.float32)
    out_ref[...] = jax.nn.gelu(z) + h_ref[...]


_fin_call = pl.pallas_call(
    _fin_body,
    grid=(10,),
    in_specs=[
        pl.BlockSpec((NC, N // 10, D), lambda i: (0, i, 0)),
        pl.BlockSpec((NC, N // 10, 16), lambda i: (0, i, 0)),
        pl.BlockSpec((N // 10, D), lambda i: (i, 0)),
        pl.BlockSpec((D, D), lambda i: (0, 0)),
        pl.BlockSpec((16, D), lambda i: (0, 0)),
    ],
    out_specs=pl.BlockSpec((N // 10, D), lambda i: (i, 0)),
    out_shape=jax.ShapeDtypeStruct((N, D), jnp.float32),
)


@functools.partial(
    pl.kernel,
    out_type=jax.ShapeDtypeStruct((G, D), jnp.float32),
    mesh=_mesh,
    scratch_types=[
        pltpu.VMEM((GP,), jnp.int32),
        pltpu.VMEM((GP,), jnp.int32),
        pltpu.VMEM((GP,), jnp.int32),
        pltpu.VMEM((GP, D), jnp.float32),
        pltpu.SemaphoreType.DMA,
    ],
)
def _gather_kernel(h_hbm, s_hbm, y_hbm, out_hbm, s_v, y_v, idx_v, rows_v, sem):
    cid = lax.axis_index("c")
    sid = lax.axis_index("s")
    lanes = lax.iota(jnp.int32, 16)
    last = jnp.full((16,), 15, jnp.int32)

    def _pscan(v):
        for k in (1, 2, 4, 8):
            sh = _shuf(v, (lanes - k) & 15)
            v = v + jnp.where(lanes >= k, sh, 0)
        return v

    @pl.when(jnp.logical_and(cid == 0, sid == 0))
    def _():
        pltpu.sync_copy(s_hbm, s_v)
        pltpu.sync_copy(y_hbm, y_v)
        totv = jnp.zeros((16,), jnp.int32)
        for g in range(GP // 16):
            sv = s_v[pl.ds(g * 16, 16)]
            cs = _pscan(sv)
            yv = y_v[pl.ds(g * 16, 16)]
            idx_v[pl.ds(g * 16, 16)] = cs - sv + totv + yv
            totv = totv + _shuf(cs, last)
        pltpu.async_copy(h_hbm.at[idx_v], rows_v, sem).wait()
        pltpu.sync_copy(rows_v.at[pl.ds(0, G)], out_hbm)


def _layer(h, src, dst, etype, ev_flat, rel_pad, Wqp, Wkv, Wo, em):
    qs_t, p_t, k_t, v_t = _prep_call(h, Wqp, Wkv)
    dn, exf = _score_kernel(qs_t, p_t, k_t, dst, src, etype, ev_flat, rel_pad)
    u = _agg_kernel(v_t, dst, src, exf)
    return _fin_call(u, dn, h, Wo, em)


def kernel(x, edge_index, edge_type, edge_vector, y, s,
           Wq0, Wk0, Wv0, We0, rel0, Wo0,
           Wq1, Wk1, Wv1, We1, rel1, Wo1):
    src = edge_index[0].astype(jnp.int32)
    dst = edge_index[1].astype(jnp.int32)
    etype = edge_type.astype(jnp.int32)

    inv = 1.0 / jnp.sqrt(jnp.float32(DH))
    em = jnp.kron(jnp.eye(NH, dtype=jnp.float32),
                  jnp.concatenate([jnp.ones((1, DH), jnp.float32),
                                   jnp.zeros((1, DH), jnp.float32)], axis=0))

    def _prep_w(Wq, We):
        wqs = Wq * inv
        blocks = jnp.transpose(We.reshape(DH, NH, DH), (1, 2, 0))
        b = jax.scipy.linalg.block_diag(*[blocks[h] for h in range(NH)])
        return jnp.concatenate([wqs, wqs @ b], axis=1)

    def _rel_pad(rel):
        return jnp.repeat(jnp.exp(rel), 2, axis=1).reshape(-1)

    Wqp0 = _prep_w(Wq0, We0)
    Wqp1 = _prep_w(Wq1, We1)
    Wkv0 = jnp.concatenate([Wk0, Wv0], axis=1)
    Wkv1 = jnp.concatenate([Wk1, Wv1], axis=1)

    ev_flat = edge_vector.reshape(-1)
    h1 = _layer(x, src, dst, etype, ev_flat, _rel_pad(rel0), Wqp0, Wkv0, Wo0, em)
    h2 = _layer(h1, src, dst, etype, ev_flat, _rel_pad(rel1), Wqp1, Wkv1, Wo1, em)

    s_p = jnp.pad(s.astype(jnp.int32), (0, GP - G))
    y_p = jnp.pad(y.astype(jnp.int32), (0, GP - G))
    return _gather_kernel(h2, s_p, y_p)
